# Initial kernel scaffold; baseline (speedup 1.0000x reference)
import jax
import jax.numpy as jnp
from jax.experimental import pallas as pl


def kernel(x, norm_plt, cls_label, conv1_w, conv1_b, sn_w1, sn_b1, sn_w2, sn_b2, mats, convt_w):
    raise NotImplementedError("write your pallas kernel here")


# speedup vs baseline: 4.2717x; 4.2717x over previous
import functools

import jax
import jax.numpy as jnp
from jax import lax
from jax.experimental import pallas as pl
from jax.experimental.pallas import tpu as pltpu
from jax.experimental.pallas import tpu_sc as plsc

B = 4
C = 9
N = 1024
K = 30
KP = 32
M = 8
NP = B * N
E = NP * KP
EV = B * N * K
EPS = 1e-5
NEG = -3.0e38


def _knn_body(x_ref, xt_ref, idx_ref):
    b = pl.program_id(0)
    xb = x_ref[0]
    xtb = xt_ref[0]
    gram = jnp.dot(xtb, xb, preferred_element_type=jnp.float32)
    xx = jnp.sum(xb * xb, axis=0)
    p = 2.0 * gram - xx[:, None] - xx[None, :]
    lane = lax.broadcasted_iota(jnp.int32, (N, N), 1)
    sels = []
    for _ in range(K):
        v = jnp.max(p, axis=1, keepdims=True)
        cand = jnp.where(p >= v, lane, N)
        sel = jnp.min(cand, axis=1, keepdims=True)
        sels.append(sel)
        p = jnp.where(lane == sel, NEG, p)
    idx = jnp.concatenate(sels + [sels[0], sels[0]], axis=1)
    idx_ref[0] = idx + b * N


def _knn(x, xt):
    return pl.pallas_call(
        _knn_body,
        grid=(B,),
        in_specs=[
            pl.BlockSpec((1, C, N), lambda b: (b, 0, 0)),
            pl.BlockSpec((1, N, C), lambda b: (b, 0, 0)),
        ],
        out_specs=pl.BlockSpec((1, N, KP), lambda b: (b, 0, 0)),
        out_shape=jax.ShapeDtypeStruct((B, N, KP), jnp.int32),
    )(x, xt)


_NW = 32
_CHUNK = 128


@functools.cache
def _make_gather(D):
    n_per_w = E // _NW
    n_chunks = n_per_w // _CHUNK
    mesh = plsc.VectorSubcoreMesh(core_axis_name="c", subcore_axis_name="s",
                                  num_cores=2, num_subcores=16)

    @functools.partial(
        pl.kernel,
        out_type=jax.ShapeDtypeStruct((E, D), jnp.float32),
        mesh=mesh,
        scratch_types=[
            pltpu.VMEM((_CHUNK,), jnp.int32),
            pltpu.VMEM((_CHUNK, D), jnp.float32),
            pltpu.SemaphoreType.DMA,
        ],
        compiler_params=pltpu.CompilerParams(use_tc_tiling_on_sc=False),
    )
    def gk(table_hbm, idx_hbm, out_hbm, idx_v, rows_v, sem):
        wid = lax.axis_index("s") * 2 + lax.axis_index("c")
        base = wid * n_per_w

        @pl.loop(0, n_chunks)
        def _(ci):
            off = base + ci * _CHUNK
            pltpu.sync_copy(idx_hbm.at[pl.ds(off, _CHUNK)], idx_v)
            pltpu.async_copy(table_hbm.at[idx_v], rows_v, sem).wait()
            pltpu.sync_copy(rows_v, out_hbm.at[pl.ds(off, _CHUNK)])

    return gk


def _gather16(table, gidx):
    return _make_gather(16)(table, gidx)


def _gather64(table, gidx):
    return _make_gather(64)(table, gidx)


_NCB = 16
_PCB = NP // _NCB
_ECB = E // _NCB


def _build_feat(nbr16_ref, xt_ref):
    xc = xt_ref[...]
    ctr = jnp.broadcast_to(xc[:, None, :], (_PCB, KP, C)).reshape(_ECB, C)
    nbr = nbr16_ref[:, 0:C]
    diff = nbr - ctr
    dist = jnp.sqrt(jnp.sum(diff * diff, axis=1, keepdims=True) + 1e-12)
    feat = jnp.concatenate([ctr, nbr, diff, dist], axis=1)
    krow = lax.broadcasted_iota(jnp.int32, (_ECB, 1), 0) % KP
    valid = (krow < K).astype(jnp.float32)
    return feat * valid, valid


def _stageB1_body(nbr16_ref, xt_ref, mo_ref, fsum_ref, mo_acc, fs_acc):
    c = pl.program_id(0)

    @pl.when(c == 0)
    def _():
        mo_acc[...] = jnp.zeros_like(mo_acc)
        fs_acc[...] = jnp.zeros_like(fs_acc)

    f, _ = _build_feat(nbr16_ref, xt_ref)
    mo_acc[...] += lax.dot_general(f, f, (((0,), (0,)), ((), ())),
                                   preferred_element_type=jnp.float32)
    fs_acc[...] += jnp.sum(f, axis=0, keepdims=True)
    mo_ref[...] = mo_acc[...]
    fsum_ref[...] = fs_acc[...]


def _stageB1(nbr16, xtf):
    return pl.pallas_call(
        _stageB1_body,
        grid=(_NCB,),
        in_specs=[
            pl.BlockSpec((_ECB, 16), lambda c: (c, 0)),
            pl.BlockSpec((_PCB, C), lambda c: (c, 0)),
        ],
        out_specs=[
            pl.BlockSpec((28, 28), lambda c: (0, 0)),
            pl.BlockSpec((1, 28), lambda c: (0, 0)),
        ],
        out_shape=[
            jax.ShapeDtypeStruct((28, 28), jnp.float32),
            jax.ShapeDtypeStruct((1, 28), jnp.float32),
        ],
        scratch_shapes=[
            pltpu.VMEM((28, 28), jnp.float32),
            pltpu.VMEM((1, 28), jnp.float32),
        ],
    )(nbr16, xtf)


def _stageB2_body(nbr16_ref, xt_ref, mo_ref, fsum_ref, w1_ref, b1_ref,
                  w2_ref, b2_ref, wf_ref, bc_ref, s_ref, x1_ref):
    mo = mo_ref[...]
    fmean = fsum_ref[...] / EV

    def _stats(w, bias):
        mu = jnp.dot(fmean, w, preferred_element_type=jnp.float32) + bias
        t = jnp.dot(mo, w, preferred_element_type=jnp.float32)
        e2 = jnp.sum(w * t, axis=0, keepdims=True) / EV
        e2 = e2 + 2.0 * bias * (mu - bias) + bias * bias
        var = e2 - mu * mu
        return mu, lax.rsqrt(var + EPS)

    w1 = w1_ref[...]
    b1 = b1_ref[...]
    w2 = w2_ref[...]
    b2 = b2_ref[...]
    wf = wf_ref[...]
    bc = bc_ref[...]

    f, valid = _build_feat(nbr16_ref, xt_ref)
    for i in range(4):
        mu1, is1 = _stats(w1[i], b1[i][None, :])
        h1 = jnp.dot(f, w1[i], preferred_element_type=jnp.float32)
        h1 = jnp.maximum((h1 + b1[i][None, :] - mu1) * is1, 0.0)
        p2 = jnp.dot(h1, w2[i], preferred_element_type=jnp.float32)
        p2 = p2 + b2[i][None, :]
        p2 = p2 - jnp.max(p2, axis=1, keepdims=True)
        ex = jnp.exp(p2)
        sm = ex / jnp.sum(ex, axis=1, keepdims=True)
        s_ref[:, pl.ds(8 * i, 8)] = sm * valid
    muf, isf = _stats(wf, bc)
    h = jnp.dot(f, wf, preferred_element_type=jnp.float32)
    h = jnp.maximum((h + bc - muf) * isf, 0.0)
    h = jnp.where(valid > 0.0, h, -1.0)
    x1_ref[...] = jnp.max(h.reshape(_PCB, KP, 64), axis=1)


def _stageB2(nbr16, xtf, mo, fsum, w1, b1, w2, b2, wf, bc):
    return pl.pallas_call(
        _stageB2_body,
        grid=(_NCB,),
        in_specs=[
            pl.BlockSpec((_ECB, 16), lambda c: (c, 0)),
            pl.BlockSpec((_PCB, C), lambda c: (c, 0)),
            pl.BlockSpec((28, 28), lambda c: (0, 0)),
            pl.BlockSpec((1, 28), lambda c: (0, 0)),
            pl.BlockSpec((4, 28, 16), lambda c: (0, 0, 0)),
            pl.BlockSpec((4, 16), lambda c: (0, 0)),
            pl.BlockSpec((4, 16, 8), lambda c: (0, 0, 0)),
            pl.BlockSpec((4, 8), lambda c: (0, 0)),
            pl.BlockSpec((28, 64), lambda c: (0, 0)),
            pl.BlockSpec((1, 64), lambda c: (0, 0)),
        ],
        out_specs=[
            pl.BlockSpec((_ECB, 32), lambda c: (c, 0)),
            pl.BlockSpec((_PCB, 64), lambda c: (c, 0)),
        ],
        out_shape=[
            jax.ShapeDtypeStruct((E, 32), jnp.float32),
            jax.ShapeDtypeStruct((NP, 64), jnp.float32),
        ],
    )(nbr16, xtf, mo, fsum, w1, b1, w2, b2, wf, bc)


def _stageC_body(i, g_ref, s_ref, cur_ref, wc_ref, out_ref):
    cb = pl.program_id(0)
    g = g_ref[...]
    s = s_ref[:, 8 * i:8 * i + 8]
    cur = cur_ref[pl.ds(cb * _PCB, _PCB), :]
    cols = []
    for m in range(M):
        t = g * s[:, m][:, None]
        cols.append(jnp.sum(t.reshape(_PCB, KP, 64), axis=1))
    ssum = jnp.sum(s.reshape(_PCB, KP, M), axis=1)
    for m in range(M):
        cols.append(ssum[:, m][:, None] * cur)
    xmat = jnp.concatenate(cols, axis=1)
    res = jnp.dot(xmat, wc_ref[...], preferred_element_type=jnp.float32)
    out_ref[pl.ds(cb * _PCB, _PCB), :] = res

    @pl.when(cb == _NCB - 1)
    def _():
        o = out_ref[...]
        mu = jnp.mean(o, axis=0, keepdims=True)
        var = jnp.mean((o - mu) * (o - mu), axis=0, keepdims=True)
        out_ref[...] = jnp.maximum((o - mu) * lax.rsqrt(var + EPS), 0.0)


def _stageC(g, s_all, cur, wcomb, i):
    return pl.pallas_call(
        functools.partial(_stageC_body, i),
        grid=(_NCB,),
        in_specs=[
            pl.BlockSpec((_ECB, 64), lambda cb: (cb, 0)),
            pl.BlockSpec((_ECB, 32), lambda cb: (cb, 0)),
            pl.BlockSpec((NP, 64), lambda cb: (0, 0)),
            pl.BlockSpec((1024, 64), lambda cb: (0, 0)),
        ],
        out_specs=pl.BlockSpec((NP, 64), lambda cb: (0, 0)),
        out_shape=jax.ShapeDtypeStruct((NP, 64), jnp.float32),
    )(g, s_all, cur, wcomb)


def _stageD_body(f0, f1, f2, f3, f4, wt_ref, out_ref):
    feats = jnp.concatenate(
        [f0[...], f1[...], f2[...], f3[...], f4[...]], axis=1)
    xc = jnp.dot(feats, wt_ref[...], preferred_element_type=jnp.float32)
    mu = jnp.mean(xc, axis=0, keepdims=True)
    var = jnp.mean((xc - mu) * (xc - mu), axis=0, keepdims=True)
    xn = jnp.maximum((xc - mu) * lax.rsqrt(var + EPS), 0.0)
    out_ref[...] = jnp.max(xn.reshape(B, N, 1024), axis=1)


def _stageD(feats, wt):
    return pl.pallas_call(
        _stageD_body,
        in_specs=[pl.BlockSpec(memory_space=pltpu.VMEM)] * 6,
        out_specs=pl.BlockSpec(memory_space=pltpu.VMEM),
        out_shape=jax.ShapeDtypeStruct((B, 1024), jnp.float32),
    )(*feats, wt)


def kernel(x, norm_plt, cls_label, conv1_w, conv1_b, sn_w1, sn_b1, sn_w2,
           sn_b2, mats, convt_w):
    xt = x.transpose(0, 2, 1)
    idx = _knn(x, xt)
    gidx = idx.reshape(E)
    xtf = xt.reshape(NP, C)
    xtpad = jnp.pad(xtf, ((0, 0), (0, 16 - C)))
    nbr16 = _gather16(xtpad, gidx)

    w1 = sn_w1.transpose(0, 2, 1)
    w2 = sn_w2.transpose(0, 2, 1)
    wf = jnp.zeros((28, 64), jnp.float32)
    wf = wf.at[0:9].set(conv1_w[:, 9:18].T)
    wf = wf.at[18:27].set(conv1_w[:, 0:9].T)
    mo, fsum = _stageB1(nbr16, xtf)
    s_all, x1 = _stageB2(nbr16, xtf, mo, fsum, w1, sn_b1, w2, sn_b2, wf,
                         conv1_b[None, :])

    feats = [x1]
    cur = x1
    for i in range(4):
        k1 = mats[i][:64]
        wpo = k1 + mats[i][64:]
        wpo_stack = wpo.reshape(64, M, 64).transpose(1, 0, 2).reshape(512, 64)
        k1_stack = k1.reshape(64, M, 64).transpose(1, 0, 2).reshape(512, 64)
        wcomb = jnp.concatenate([wpo_stack, -k1_stack], axis=0)
        g = _gather64(cur, gidx)
        cur = _stageC(g, s_all, cur, wcomb, i)
        feats.append(cur)

    return _stageD(feats, convt_w.T)


# speedup vs baseline: 4.2741x; 1.0006x over previous
import functools

import jax
import jax.numpy as jnp
from jax import lax
from jax.experimental import pallas as pl
from jax.experimental.pallas import tpu as pltpu
from jax.experimental.pallas import tpu_sc as plsc

B = 4
C = 9
N = 1024
K = 30
KP = 32
M = 8
NP = B * N
E = NP * KP
EV = B * N * K
EPS = 1e-5
NEG = -3.0e38


def _knn_body(x_ref, xt_ref, idx_ref):
    blocks = []
    for b in range(B):
        xb = x_ref[b]
        xtb = xt_ref[pl.ds(b * N, N), :]
        gram = jnp.dot(xtb, xb, preferred_element_type=jnp.float32)
        xx = jnp.sum(xb * xb, axis=0)
        blocks.append(2.0 * gram - xx[:, None] - xx[None, :])
    p = jnp.concatenate(blocks, axis=0)
    lane = lax.broadcasted_iota(jnp.int32, (NP, N), 1)
    sels = []
    for _ in range(K):
        v = jnp.max(p, axis=1, keepdims=True)
        cand = jnp.where(p >= v, lane, N)
        sel = jnp.min(cand, axis=1, keepdims=True)
        sels.append(sel)
        p = jnp.where(lane == sel, NEG, p)
    idx = jnp.concatenate(sels + [sels[0], sels[0]], axis=1)
    boff = lax.broadcasted_iota(jnp.int32, (NP, KP), 0) // N * N
    idx_ref[...] = idx + boff


def _knn(x, xt):
    return pl.pallas_call(
        _knn_body,
        in_specs=[
            pl.BlockSpec(memory_space=pltpu.VMEM),
            pl.BlockSpec(memory_space=pltpu.VMEM),
        ],
        out_specs=pl.BlockSpec(memory_space=pltpu.VMEM),
        out_shape=jax.ShapeDtypeStruct((NP, KP), jnp.int32),
    )(x, xt)


_NW = 32
_CHUNK = 128


@functools.cache
def _make_gather(D):
    n_per_w = E // _NW
    n_chunks = n_per_w // _CHUNK
    mesh = plsc.VectorSubcoreMesh(core_axis_name="c", subcore_axis_name="s",
                                  num_cores=2, num_subcores=16)

    @functools.partial(
        pl.kernel,
        out_type=jax.ShapeDtypeStruct((E, D), jnp.float32),
        mesh=mesh,
        scratch_types=[
            pltpu.VMEM((_CHUNK,), jnp.int32),
            pltpu.VMEM((_CHUNK, D), jnp.float32),
            pltpu.SemaphoreType.DMA,
        ],
        compiler_params=pltpu.CompilerParams(use_tc_tiling_on_sc=False),
    )
    def gk(table_hbm, idx_hbm, out_hbm, idx_v, rows_v, sem):
        wid = lax.axis_index("s") * 2 + lax.axis_index("c")
        base = wid * n_per_w

        @pl.loop(0, n_chunks)
        def _(ci):
            off = base + ci * _CHUNK
            pltpu.sync_copy(idx_hbm.at[pl.ds(off, _CHUNK)], idx_v)
            pltpu.async_copy(table_hbm.at[idx_v], rows_v, sem).wait()
            pltpu.sync_copy(rows_v, out_hbm.at[pl.ds(off, _CHUNK)])

    return gk


def _gather16(table, gidx):
    return _make_gather(16)(table, gidx)


def _gather64(table, gidx):
    return _make_gather(64)(table, gidx)


_NCB = 16
_PCB = NP // _NCB
_ECB = E // _NCB


def _build_feat(nbr16_ref, xt_ref):
    xc = xt_ref[...]
    ctr = jnp.broadcast_to(xc[:, None, :], (_PCB, KP, C)).reshape(_ECB, C)
    nbr = nbr16_ref[:, 0:C]
    diff = nbr - ctr
    dist = jnp.sqrt(jnp.sum(diff * diff, axis=1, keepdims=True) + 1e-12)
    feat = jnp.concatenate([ctr, nbr, diff, dist], axis=1)
    krow = lax.broadcasted_iota(jnp.int32, (_ECB, 1), 0) % KP
    valid = (krow < K).astype(jnp.float32)
    return feat * valid, valid


def _stageB1_body(nbr16_ref, xt_ref, mo_ref, fsum_ref, mo_acc, fs_acc):
    c = pl.program_id(0)

    @pl.when(c == 0)
    def _():
        mo_acc[...] = jnp.zeros_like(mo_acc)
        fs_acc[...] = jnp.zeros_like(fs_acc)

    f, _ = _build_feat(nbr16_ref, xt_ref)
    mo_acc[...] += lax.dot_general(f, f, (((0,), (0,)), ((), ())),
                                   preferred_element_type=jnp.float32)
    fs_acc[...] += jnp.sum(f, axis=0, keepdims=True)
    mo_ref[...] = mo_acc[...]
    fsum_ref[...] = fs_acc[...]


def _stageB1(nbr16, xtf):
    return pl.pallas_call(
        _stageB1_body,
        grid=(_NCB,),
        in_specs=[
            pl.BlockSpec((_ECB, 16), lambda c: (c, 0)),
            pl.BlockSpec((_PCB, C), lambda c: (c, 0)),
        ],
        out_specs=[
            pl.BlockSpec((28, 28), lambda c: (0, 0)),
            pl.BlockSpec((1, 28), lambda c: (0, 0)),
        ],
        out_shape=[
            jax.ShapeDtypeStruct((28, 28), jnp.float32),
            jax.ShapeDtypeStruct((1, 28), jnp.float32),
        ],
        scratch_shapes=[
            pltpu.VMEM((28, 28), jnp.float32),
            pltpu.VMEM((1, 28), jnp.float32),
        ],
    )(nbr16, xtf)


def _stageB2_body(nbr16_ref, xt_ref, mo_ref, fsum_ref, w1_ref, b1_ref,
                  w2_ref, b2_ref, wf_ref, bc_ref, s_ref, x1_ref):
    mo = mo_ref[...]
    fmean = fsum_ref[...] / EV

    def _stats(w, bias):
        mu = jnp.dot(fmean, w, preferred_element_type=jnp.float32) + bias
        t = jnp.dot(mo, w, preferred_element_type=jnp.float32)
        e2 = jnp.sum(w * t, axis=0, keepdims=True) / EV
        e2 = e2 + 2.0 * bias * (mu - bias) + bias * bias
        var = e2 - mu * mu
        return mu, lax.rsqrt(var + EPS)

    w1 = w1_ref[...]
    b1 = b1_ref[...]
    w2 = w2_ref[...]
    b2 = b2_ref[...]
    wf = wf_ref[...]
    bc = bc_ref[...]

    f, valid = _build_feat(nbr16_ref, xt_ref)
    for i in range(4):
        mu1, is1 = _stats(w1[i], b1[i][None, :])
        h1 = jnp.dot(f, w1[i], preferred_element_type=jnp.float32)
        h1 = jnp.maximum((h1 + b1[i][None, :] - mu1) * is1, 0.0)
        p2 = jnp.dot(h1, w2[i], preferred_element_type=jnp.float32)
        p2 = p2 + b2[i][None, :]
        p2 = p2 - jnp.max(p2, axis=1, keepdims=True)
        ex = jnp.exp(p2)
        sm = ex / jnp.sum(ex, axis=1, keepdims=True)
        s_ref[:, pl.ds(8 * i, 8)] = sm * valid
    muf, isf = _stats(wf, bc)
    h = jnp.dot(f, wf, preferred_element_type=jnp.float32)
    h = jnp.maximum((h + bc - muf) * isf, 0.0)
    h = jnp.where(valid > 0.0, h, -1.0)
    x1_ref[...] = jnp.max(h.reshape(_PCB, KP, 64), axis=1)


def _stageB2(nbr16, xtf, mo, fsum, w1, b1, w2, b2, wf, bc):
    return pl.pallas_call(
        _stageB2_body,
        grid=(_NCB,),
        in_specs=[
            pl.BlockSpec((_ECB, 16), lambda c: (c, 0)),
            pl.BlockSpec((_PCB, C), lambda c: (c, 0)),
            pl.BlockSpec((28, 28), lambda c: (0, 0)),
            pl.BlockSpec((1, 28), lambda c: (0, 0)),
            pl.BlockSpec((4, 28, 16), lambda c: (0, 0, 0)),
            pl.BlockSpec((4, 16), lambda c: (0, 0)),
            pl.BlockSpec((4, 16, 8), lambda c: (0, 0, 0)),
            pl.BlockSpec((4, 8), lambda c: (0, 0)),
            pl.BlockSpec((28, 64), lambda c: (0, 0)),
            pl.BlockSpec((1, 64), lambda c: (0, 0)),
        ],
        out_specs=[
            pl.BlockSpec((_ECB, 32), lambda c: (c, 0)),
            pl.BlockSpec((_PCB, 64), lambda c: (c, 0)),
        ],
        out_shape=[
            jax.ShapeDtypeStruct((E, 32), jnp.float32),
            jax.ShapeDtypeStruct((NP, 64), jnp.float32),
        ],
    )(nbr16, xtf, mo, fsum, w1, b1, w2, b2, wf, bc)


def _stageC_body(i, g_ref, s_ref, cur_ref, wc_ref, out_ref):
    cb = pl.program_id(0)
    g = g_ref[...]
    s = s_ref[:, 8 * i:8 * i + 8]
    cur = cur_ref[pl.ds(cb * _PCB, _PCB), :]
    cols = []
    for m in range(M):
        t = g * s[:, m][:, None]
        cols.append(jnp.sum(t.reshape(_PCB, KP, 64), axis=1))
    ssum = jnp.sum(s.reshape(_PCB, KP, M), axis=1)
    for m in range(M):
        cols.append(ssum[:, m][:, None] * cur)
    xmat = jnp.concatenate(cols, axis=1)
    res = jnp.dot(xmat, wc_ref[...], preferred_element_type=jnp.float32)
    out_ref[pl.ds(cb * _PCB, _PCB), :] = res

    @pl.when(cb == _NCB - 1)
    def _():
        o = out_ref[...]
        mu = jnp.mean(o, axis=0, keepdims=True)
        var = jnp.mean((o - mu) * (o - mu), axis=0, keepdims=True)
        out_ref[...] = jnp.maximum((o - mu) * lax.rsqrt(var + EPS), 0.0)


def _stageC(g, s_all, cur, wcomb, i):
    return pl.pallas_call(
        functools.partial(_stageC_body, i),
        grid=(_NCB,),
        in_specs=[
            pl.BlockSpec((_ECB, 64), lambda cb: (cb, 0)),
            pl.BlockSpec((_ECB, 32), lambda cb: (cb, 0)),
            pl.BlockSpec((NP, 64), lambda cb: (0, 0)),
            pl.BlockSpec((1024, 64), lambda cb: (0, 0)),
        ],
        out_specs=pl.BlockSpec((NP, 64), lambda cb: (0, 0)),
        out_shape=jax.ShapeDtypeStruct((NP, 64), jnp.float32),
    )(g, s_all, cur, wcomb)


def _stageD_body(f0, f1, f2, f3, f4, wt_ref, out_ref):
    feats = jnp.concatenate(
        [f0[...], f1[...], f2[...], f3[...], f4[...]], axis=1)
    xc = jnp.dot(feats, wt_ref[...], preferred_element_type=jnp.float32)
    mu = jnp.mean(xc, axis=0, keepdims=True)
    var = jnp.mean((xc - mu) * (xc - mu), axis=0, keepdims=True)
    xn = jnp.maximum((xc - mu) * lax.rsqrt(var + EPS), 0.0)
    out_ref[...] = jnp.max(xn.reshape(B, N, 1024), axis=1)


def _stageD(feats, wt):
    return pl.pallas_call(
        _stageD_body,
        in_specs=[pl.BlockSpec(memory_space=pltpu.VMEM)] * 6,
        out_specs=pl.BlockSpec(memory_space=pltpu.VMEM),
        out_shape=jax.ShapeDtypeStruct((B, 1024), jnp.float32),
    )(*feats, wt)


def kernel(x, norm_plt, cls_label, conv1_w, conv1_b, sn_w1, sn_b1, sn_w2,
           sn_b2, mats, convt_w):
    xt = x.transpose(0, 2, 1)
    xtf = xt.reshape(NP, C)
    idx = _knn(x, xtf)
    gidx = idx.reshape(E)
    xtpad = jnp.pad(xtf, ((0, 0), (0, 16 - C)))
    nbr16 = _gather16(xtpad, gidx)

    w1 = sn_w1.transpose(0, 2, 1)
    w2 = sn_w2.transpose(0, 2, 1)
    wf = jnp.zeros((28, 64), jnp.float32)
    wf = wf.at[0:9].set(conv1_w[:, 9:18].T)
    wf = wf.at[18:27].set(conv1_w[:, 0:9].T)
    mo, fsum = _stageB1(nbr16, xtf)
    s_all, x1 = _stageB2(nbr16, xtf, mo, fsum, w1, sn_b1, w2, sn_b2, wf,
                         conv1_b[None, :])

    feats = [x1]
    cur = x1
    for i in range(4):
        k1 = mats[i][:64]
        wpo = k1 + mats[i][64:]
        wpo_stack = wpo.reshape(64, M, 64).transpose(1, 0, 2).reshape(512, 64)
        k1_stack = k1.reshape(64, M, 64).transpose(1, 0, 2).reshape(512, 64)
        wcomb = jnp.concatenate([wpo_stack, -k1_stack], axis=0)
        g = _gather64(cur, gidx)
        cur = _stageC(g, s_all, cur, wcomb, i)
        feats.append(cur)

    return _stageD(feats, convt_w.T)


# speedup vs baseline: 4.9387x; 1.1555x over previous
import functools

import jax
import jax.numpy as jnp
from jax import lax
from jax.experimental import pallas as pl
from jax.experimental.pallas import tpu as pltpu
from jax.experimental.pallas import tpu_sc as plsc

B = 4
C = 9
N = 1024
K = 30
KP = 32
M = 8
NP = B * N
E = NP * KP
EV = B * N * K
EPS = 1e-5
NEG = -3.0e38


def _knn_body(x_ref, xt_ref, idx_ref):
    blocks = []
    for b in range(B):
        xb = x_ref[b]
        xtb = xt_ref[pl.ds(b * N, N), :]
        gram = jnp.dot(xtb, xb, preferred_element_type=jnp.float32)
        xx = jnp.sum(xb * xb, axis=0)
        blocks.append(2.0 * gram - xx[:, None] - xx[None, :])
    p = jnp.concatenate(blocks, axis=0)
    lane = lax.broadcasted_iota(jnp.int32, (NP, N), 1)
    sels = []
    for _ in range(K):
        v = jnp.max(p, axis=1, keepdims=True)
        cand = jnp.where(p >= v, lane, N)
        sel = jnp.min(cand, axis=1, keepdims=True)
        sels.append(sel)
        p = jnp.where(lane == sel, NEG, p)
    idx = jnp.concatenate(sels + [sels[0], sels[0]], axis=1)
    boff = lax.broadcasted_iota(jnp.int32, (NP, KP), 0) // N * N
    idx_ref[...] = idx + boff


def _knn(x, xt):
    return pl.pallas_call(
        _knn_body,
        in_specs=[
            pl.BlockSpec(memory_space=pltpu.VMEM),
            pl.BlockSpec(memory_space=pltpu.VMEM),
        ],
        out_specs=pl.BlockSpec(memory_space=pltpu.VMEM),
        out_shape=jax.ShapeDtypeStruct((NP, KP), jnp.int32),
    )(x, xt)


_NW = 32
_CHUNK = 128
_SUP = 512


@functools.cache
def _make_gather128():
    n_per_w = E // _NW
    n_sup = n_per_w // _SUP
    n_fire = _SUP // _CHUNK
    mesh = plsc.VectorSubcoreMesh(core_axis_name="c", subcore_axis_name="s",
                                  num_cores=2, num_subcores=16)

    @functools.partial(
        pl.kernel,
        out_type=jax.ShapeDtypeStruct((E, 128), jnp.float32),
        mesh=mesh,
        scratch_types=[
            pltpu.VMEM((n_per_w,), jnp.int32),
            pltpu.VMEM((_SUP, 128), jnp.float32),
            pltpu.SemaphoreType.DMA,
        ],
        compiler_params=pltpu.CompilerParams(use_tc_tiling_on_sc=False),
    )
    def gk(table_hbm, idx_hbm, out_hbm, idx_v, rows_v, sem):
        wid = lax.axis_index("s") * 2 + lax.axis_index("c")
        base = wid * n_per_w
        pltpu.sync_copy(idx_hbm.at[pl.ds(base, n_per_w)], idx_v)

        @pl.loop(0, n_sup)
        def _(si):
            copies = [
                pltpu.async_copy(
                    table_hbm.at[idx_v.at[pl.ds(si * _SUP + f * _CHUNK,
                                                _CHUNK)]],
                    rows_v.at[pl.ds(f * _CHUNK, _CHUNK)], sem)
                for f in range(n_fire)
            ]
            for cp in copies:
                cp.wait()
            pltpu.sync_copy(rows_v, out_hbm.at[pl.ds(base + si * _SUP, _SUP)])

    return gk


def _gather128(table, gidx):
    return _make_gather128()(table, gidx)


_NCB = 16
_PCB = NP // _NCB
_ECB = E // _NCB


def _build_feat(nbr_ref, xt_ref):
    xc = xt_ref[...]
    ctr = jnp.broadcast_to(xc[:, None, :], (_PCB, KP, C)).reshape(_ECB, C)
    nbr = nbr_ref[:, 0:C]
    diff = nbr - ctr
    dist = jnp.sqrt(jnp.sum(diff * diff, axis=1, keepdims=True) + 1e-12)
    feat = jnp.concatenate([ctr, nbr, diff, dist], axis=1)
    krow = lax.broadcasted_iota(jnp.int32, (_ECB, 1), 0) % KP
    valid = (krow < K).astype(jnp.float32)
    return feat * valid, valid


def _stageB1_body(nbr16_ref, xt_ref, mo_ref, fsum_ref, mo_acc, fs_acc):
    c = pl.program_id(0)

    @pl.when(c == 0)
    def _():
        mo_acc[...] = jnp.zeros_like(mo_acc)
        fs_acc[...] = jnp.zeros_like(fs_acc)

    f, _ = _build_feat(nbr16_ref, xt_ref)
    mo_acc[...] += lax.dot_general(f, f, (((0,), (0,)), ((), ())),
                                   preferred_element_type=jnp.float32)
    fs_acc[...] += jnp.sum(f, axis=0, keepdims=True)
    mo_ref[...] = mo_acc[...]
    fsum_ref[...] = fs_acc[...]


def _stageB1(nbr16, xtf):
    return pl.pallas_call(
        _stageB1_body,
        grid=(_NCB,),
        in_specs=[
            pl.BlockSpec((_ECB, 128), lambda c: (c, 0)),
            pl.BlockSpec((_PCB, C), lambda c: (c, 0)),
        ],
        out_specs=[
            pl.BlockSpec((28, 28), lambda c: (0, 0)),
            pl.BlockSpec((1, 28), lambda c: (0, 0)),
        ],
        out_shape=[
            jax.ShapeDtypeStruct((28, 28), jnp.float32),
            jax.ShapeDtypeStruct((1, 28), jnp.float32),
        ],
        scratch_shapes=[
            pltpu.VMEM((28, 28), jnp.float32),
            pltpu.VMEM((1, 28), jnp.float32),
        ],
    )(nbr16, xtf)


def _stageB2_body(nbr16_ref, xt_ref, mo_ref, fsum_ref, w1_ref, b1_ref,
                  w2_ref, b2_ref, wf_ref, bc_ref, s_ref, x1_ref):
    mo = mo_ref[...]
    fmean = fsum_ref[...] / EV

    def _stats(w, bias):
        mu = jnp.dot(fmean, w, preferred_element_type=jnp.float32) + bias
        t = jnp.dot(mo, w, preferred_element_type=jnp.float32)
        e2 = jnp.sum(w * t, axis=0, keepdims=True) / EV
        e2 = e2 + 2.0 * bias * (mu - bias) + bias * bias
        var = e2 - mu * mu
        return mu, lax.rsqrt(var + EPS)

    w1 = w1_ref[...]
    b1 = b1_ref[...]
    w2 = w2_ref[...]
    b2 = b2_ref[...]
    wf = wf_ref[...]
    bc = bc_ref[...]

    f, valid = _build_feat(nbr16_ref, xt_ref)
    for i in range(4):
        mu1, is1 = _stats(w1[i], b1[i][None, :])
        h1 = jnp.dot(f, w1[i], preferred_element_type=jnp.float32)
        h1 = jnp.maximum((h1 + b1[i][None, :] - mu1) * is1, 0.0)
        p2 = jnp.dot(h1, w2[i], preferred_element_type=jnp.float32)
        p2 = p2 + b2[i][None, :]
        p2 = p2 - jnp.max(p2, axis=1, keepdims=True)
        ex = jnp.exp(p2)
        sm = ex / jnp.sum(ex, axis=1, keepdims=True)
        s_ref[:, pl.ds(8 * i, 8)] = sm * valid
    muf, isf = _stats(wf, bc)
    h = jnp.dot(f, wf, preferred_element_type=jnp.float32)
    h = jnp.maximum((h + bc - muf) * isf, 0.0)
    h = jnp.where(valid > 0.0, h, -1.0)
    x1 = jnp.max(h.reshape(_PCB, KP, 64), axis=1)
    x1_ref[...] = jnp.concatenate(
        [x1, jnp.zeros((_PCB, 64), jnp.float32)], axis=1)


def _stageB2(nbr16, xtf, mo, fsum, w1, b1, w2, b2, wf, bc):
    return pl.pallas_call(
        _stageB2_body,
        grid=(_NCB,),
        in_specs=[
            pl.BlockSpec((_ECB, 128), lambda c: (c, 0)),
            pl.BlockSpec((_PCB, C), lambda c: (c, 0)),
            pl.BlockSpec((28, 28), lambda c: (0, 0)),
            pl.BlockSpec((1, 28), lambda c: (0, 0)),
            pl.BlockSpec((4, 28, 16), lambda c: (0, 0, 0)),
            pl.BlockSpec((4, 16), lambda c: (0, 0)),
            pl.BlockSpec((4, 16, 8), lambda c: (0, 0, 0)),
            pl.BlockSpec((4, 8), lambda c: (0, 0)),
            pl.BlockSpec((28, 64), lambda c: (0, 0)),
            pl.BlockSpec((1, 64), lambda c: (0, 0)),
        ],
        out_specs=[
            pl.BlockSpec((_ECB, 32), lambda c: (c, 0)),
            pl.BlockSpec((_PCB, 128), lambda c: (c, 0)),
        ],
        out_shape=[
            jax.ShapeDtypeStruct((E, 32), jnp.float32),
            jax.ShapeDtypeStruct((NP, 128), jnp.float32),
        ],
    )(nbr16, xtf, mo, fsum, w1, b1, w2, b2, wf, bc)


def _stageC_body(i, g_ref, s_ref, cur_ref, wc_ref, out_ref):
    cb = pl.program_id(0)
    g = g_ref[:, 0:64]
    s = s_ref[:, 8 * i:8 * i + 8]
    cur = cur_ref[pl.ds(cb * _PCB, _PCB), 0:64]
    cols = []
    for m in range(M):
        t = g * s[:, m][:, None]
        cols.append(jnp.sum(t.reshape(_PCB, KP, 64), axis=1))
    ssum = jnp.sum(s.reshape(_PCB, KP, M), axis=1)
    for m in range(M):
        cols.append(ssum[:, m][:, None] * cur)
    xmat = jnp.concatenate(cols, axis=1)
    res = jnp.dot(xmat, wc_ref[...], preferred_element_type=jnp.float32)
    out_ref[pl.ds(cb * _PCB, _PCB), :] = jnp.concatenate(
        [res, jnp.zeros((_PCB, 64), jnp.float32)], axis=1)

    @pl.when(cb == _NCB - 1)
    def _():
        o = out_ref[:, 0:64]
        mu = jnp.mean(o, axis=0, keepdims=True)
        var = jnp.mean((o - mu) * (o - mu), axis=0, keepdims=True)
        out_ref[:, 0:64] = jnp.maximum((o - mu) * lax.rsqrt(var + EPS), 0.0)


def _stageC(g, s_all, cur, wcomb, i):
    return pl.pallas_call(
        functools.partial(_stageC_body, i),
        grid=(_NCB,),
        in_specs=[
            pl.BlockSpec((_ECB, 128), lambda cb: (cb, 0)),
            pl.BlockSpec((_ECB, 32), lambda cb: (cb, 0)),
            pl.BlockSpec((NP, 128), lambda cb: (0, 0)),
            pl.BlockSpec((1024, 64), lambda cb: (0, 0)),
        ],
        out_specs=pl.BlockSpec((NP, 128), lambda cb: (0, 0)),
        out_shape=jax.ShapeDtypeStruct((NP, 128), jnp.float32),
    )(g, s_all, cur, wcomb)


def _stageD_body(f0, f1, f2, f3, f4, wt_ref, out_ref):
    feats = jnp.concatenate(
        [f0[:, 0:64], f1[:, 0:64], f2[:, 0:64], f3[:, 0:64], f4[:, 0:64]],
        axis=1)
    xc = jnp.dot(feats, wt_ref[...], preferred_element_type=jnp.float32)
    mu = jnp.mean(xc, axis=0, keepdims=True)
    var = jnp.mean((xc - mu) * (xc - mu), axis=0, keepdims=True)
    xn = jnp.maximum((xc - mu) * lax.rsqrt(var + EPS), 0.0)
    out_ref[...] = jnp.max(xn.reshape(B, N, 1024), axis=1)


def _stageD(feats, wt):
    return pl.pallas_call(
        _stageD_body,
        in_specs=[pl.BlockSpec(memory_space=pltpu.VMEM)] * 6,
        out_specs=pl.BlockSpec(memory_space=pltpu.VMEM),
        out_shape=jax.ShapeDtypeStruct((B, 1024), jnp.float32),
    )(*feats, wt)


def kernel(x, norm_plt, cls_label, conv1_w, conv1_b, sn_w1, sn_b1, sn_w2,
           sn_b2, mats, convt_w):
    xt = x.transpose(0, 2, 1)
    xtf = xt.reshape(NP, C)
    idx = _knn(x, xtf)
    gidx = idx.reshape(E)
    xtpad = jnp.pad(xtf, ((0, 0), (0, 128 - C)))
    nbr = _gather128(xtpad, gidx)

    w1 = sn_w1.transpose(0, 2, 1)
    w2 = sn_w2.transpose(0, 2, 1)
    wf = jnp.zeros((28, 64), jnp.float32)
    wf = wf.at[0:9].set(conv1_w[:, 9:18].T)
    wf = wf.at[18:27].set(conv1_w[:, 0:9].T)
    mo, fsum = _stageB1(nbr, xtf)
    s_all, x1 = _stageB2(nbr, xtf, mo, fsum, w1, sn_b1, w2, sn_b2, wf,
                         conv1_b[None, :])

    feats = [x1]
    cur = x1
    for i in range(4):
        k1 = mats[i][:64]
        wpo = k1 + mats[i][64:]
        wpo_stack = wpo.reshape(64, M, 64).transpose(1, 0, 2).reshape(512, 64)
        k1_stack = k1.reshape(64, M, 64).transpose(1, 0, 2).reshape(512, 64)
        wcomb = jnp.concatenate([wpo_stack, -k1_stack], axis=0)
        g = _gather128(cur, gidx)
        cur = _stageC(g, s_all, cur, wcomb, i)
        feats.append(cur)

    return _stageD(feats, convt_w.T)
